# Initial kernel scaffold; baseline (speedup 1.0000x reference)
import jax
import jax.numpy as jnp
from jax.experimental import pallas as pl


def kernel(x, emb, W, b):
    raise NotImplementedError("write your pallas kernel here")


# speedup vs baseline: 1.6399x; 1.6399x over previous
import functools

import jax
import jax.numpy as jnp
from jax import lax
from jax.experimental import pallas as pl
from jax.experimental.pallas import tpu as pltpu
from jax.experimental.pallas import tpu_sc as plsc

VOCAB = 352899
EMB = 50
EMBP = 56
OUT = 1024
TOKENS = 26 * 4096

NC = 2
NS = 16
NW = NC * NS

B_PER_W = TOKENS // NW
CHUNK = 128
NCH = B_PER_W // CHUNK
GROUP = 13
N_GROUPS = NCH // GROUP


def _make_sc_gather():
    mesh = plsc.VectorSubcoreMesh(core_axis_name="c", subcore_axis_name="s")

    @functools.partial(
        pl.kernel,
        mesh=mesh,
        out_type=jax.ShapeDtypeStruct((NW, N_GROUPS, GROUP, CHUNK, EMBP),
                                      jnp.float32),
        scratch_types=[
            pltpu.VMEM((NCH, CHUNK), jnp.int32),
            pltpu.VMEM((GROUP, CHUNK, EMBP), jnp.float32),
            pltpu.SemaphoreType.DMA,
        ],
        compiler_params=pltpu.CompilerParams(use_tc_tiling_on_sc=False),
    )
    def gather_kernel(table_hbm, idx_hbm, out_hbm, idx_v, rows_v, sem):
        wid = lax.axis_index("s") * NC + lax.axis_index("c")
        pltpu.sync_copy(idx_hbm.at[wid], idx_v)
        for g in range(N_GROUPS):
            handles = []
            for j in range(GROUP):
                handles.append(
                    pltpu.async_copy(
                        table_hbm.at[idx_v.at[g * GROUP + j]],
                        rows_v.at[j],
                        sem,
                    )
                )
            for h in handles:
                h.wait()
            pltpu.sync_copy(rows_v, out_hbm.at[wid, g])

    return gather_kernel


_sc_gather = _make_sc_gather()


BT = 1024


def _matmul_body(v_ref, wt_ref, b_ref, o_ref):
    o_ref[...] = (
        jnp.dot(v_ref[...], wt_ref[...], preferred_element_type=jnp.float32)
        + b_ref[...]
    )


def _tc_project(v, wt, b2d):
    grid = (TOKENS // BT,)
    return pl.pallas_call(
        _matmul_body,
        grid=grid,
        in_specs=[
            pl.BlockSpec((BT, EMBP), lambda i: (i, 0)),
            pl.BlockSpec((EMBP, OUT), lambda i: (0, 0)),
            pl.BlockSpec((1, OUT), lambda i: (0, 0)),
        ],
        out_specs=pl.BlockSpec((BT, OUT), lambda i: (i, 0)),
        out_shape=jax.ShapeDtypeStruct((TOKENS, OUT), jnp.float32),
        compiler_params=pltpu.CompilerParams(
            dimension_semantics=("arbitrary",),
        ),
    )(v, wt, b2d)


def kernel(x, emb, W, b):
    idx = jnp.transpose(x, (1, 0)).reshape(NW, NCH, CHUNK).astype(jnp.int32)
    emb_p = jnp.pad(emb, ((0, 0), (0, EMBP - EMB)))
    v = _sc_gather(emb_p, idx).reshape(TOKENS, EMBP)
    wt_p = jnp.pad(W.T, ((0, EMBP - EMB), (0, 0)))
    y = _tc_project(v, wt_p, b.reshape(1, OUT))
    return y.reshape(4096, 26, OUT)


# speedup vs baseline: 1.9430x; 1.1848x over previous
import functools

import jax
import jax.numpy as jnp
from jax import lax
from jax.experimental import pallas as pl
from jax.experimental.pallas import tpu as pltpu
from jax.experimental.pallas import tpu_sc as plsc

VOCAB = 352899
EMB = 50
EMBP = 128
OUT = 1024
TOKENS = 26 * 4096

NC = 2
NS = 16
NW = NC * NS

B_PER_W = TOKENS // NW
CHUNK = 128
NCH = B_PER_W // CHUNK
NBUF = 2


def _make_sc_gather():
    mesh = plsc.VectorSubcoreMesh(core_axis_name="c", subcore_axis_name="s")

    @functools.partial(
        pl.kernel,
        mesh=mesh,
        out_type=jax.ShapeDtypeStruct((NW, NCH, CHUNK, EMBP), jnp.float32),
        scratch_types=[
            pltpu.VMEM((NCH, CHUNK), jnp.int32),
            pltpu.VMEM((NBUF, CHUNK, EMBP), jnp.float32),
            [pltpu.SemaphoreType.DMA] * NBUF,
        ],
    )
    def gather_kernel(table_hbm, idx_hbm, out_hbm, idx_v, rows_v, sems):
        wid = lax.axis_index("s") * NC + lax.axis_index("c")
        pltpu.sync_copy(idx_hbm.at[wid], idx_v)

        def start(j, b):
            pltpu.async_copy(table_hbm.at[idx_v.at[j]], rows_v.at[b], sems[b])

        def drain(j, b):
            pltpu.make_async_copy(
                table_hbm.at[idx_v.at[j]], rows_v.at[b], sems[b]
            ).wait()
            pltpu.sync_copy(rows_v.at[b], out_hbm.at[wid, j])

        start(0, 0)

        def body(i, _):
            j = i * NBUF
            start(j + 1, 1)
            drain(j, 0)
            @pl.when(j + 2 < NCH)
            def _():
                start(j + 2, 0)
            drain(j + 1, 1)
            return ()

        lax.fori_loop(0, NCH // NBUF, body, ())

    return gather_kernel


_sc_gather = _make_sc_gather()


BT = 1024


def _matmul_body(v_ref, wt_ref, b_ref, o_ref):
    o_ref[...] = (
        jnp.dot(v_ref[...], wt_ref[...], preferred_element_type=jnp.float32)
        + b_ref[...]
    )


def _tc_project(v, wt, b2d):
    grid = (TOKENS // BT,)
    return pl.pallas_call(
        _matmul_body,
        grid=grid,
        in_specs=[
            pl.BlockSpec((BT, EMBP), lambda i: (i, 0)),
            pl.BlockSpec((EMBP, OUT), lambda i: (0, 0)),
            pl.BlockSpec((1, OUT), lambda i: (0, 0)),
        ],
        out_specs=pl.BlockSpec((BT, OUT), lambda i: (i, 0)),
        out_shape=jax.ShapeDtypeStruct((TOKENS, OUT), jnp.float32),
        compiler_params=pltpu.CompilerParams(
            dimension_semantics=("arbitrary",),
        ),
    )(v, wt, b2d)


def kernel(x, emb, W, b):
    idx = jnp.transpose(x, (1, 0)).reshape(NW, NCH, CHUNK).astype(jnp.int32)
    emb_p = jnp.pad(emb, ((0, 0), (0, EMBP - EMB)))
    v = _sc_gather(emb_p, idx).reshape(TOKENS, EMBP)
    wt_p = jnp.pad(W.T, ((0, EMBP - EMB), (0, 0)))
    y = _tc_project(v, wt_p, b.reshape(1, OUT))
    return y.reshape(4096, 26, OUT)
